# baseline (speedup 1.0000x reference)
import functools

import jax
import jax.numpy as jnp
from jax.experimental import pallas as pl
from jax.experimental.pallas import tpu as pltpu

TEMPERATURE = 0.02
FILTER_THRESHOLD = 0.95
FILTER_FACTOR = 0.5
SCALE = 1.0 / TEMPERATURE


def _body(q_ref, d_ref, dpos_ref, out_ref,
          pos_ref, m_ref, l_ref, *, n_col_blocks, b_rows):
    c = pl.program_id(0)

    @pl.when(c == 0)
    def _init():
        pos_ref[...] = (
            jnp.sum(q_ref[...].astype(jnp.float32)
                    * dpos_ref[...].astype(jnp.float32),
                    axis=1, keepdims=True) * SCALE
        )
        m_ref[...] = jnp.full((b_rows, 1), -jnp.inf, dtype=jnp.float32)
        l_ref[...] = jnp.zeros((b_rows, 1), dtype=jnp.float32)

    s = jax.lax.dot_general(
        q_ref[...], d_ref[...],
        dimension_numbers=(((1,), (1,)), ((), ())),
        preferred_element_type=jnp.float32,
    ) * SCALE

    thresh = FILTER_THRESHOLD * pos_ref[...]
    s = jnp.where(s > thresh, s * FILTER_FACTOR, s)

    m_prev = m_ref[...]
    m_cur = jnp.maximum(m_prev, jnp.max(s, axis=1, keepdims=True))
    l_ref[...] = (
        l_ref[...] * jnp.exp(m_prev - m_cur)
        + jnp.sum(jnp.exp(s - m_cur), axis=1, keepdims=True)
    )
    m_ref[...] = m_cur

    @pl.when(c == n_col_blocks - 1)
    def _final():
        pos = pos_ref[...]
        m_run = m_ref[...]
        l_run = l_ref[...]
        m_true = jnp.maximum(m_run, pos)
        corr = jnp.where(
            pos > 0.0,
            jnp.exp(pos - m_true) - jnp.exp(FILTER_FACTOR * pos - m_true),
            0.0,
        )
        l_true = l_run * jnp.exp(m_run - m_true) + corr
        lse = m_true + jnp.log(l_true)
        out_ref[...] = jnp.reshape(-jnp.sum(pos - lse) / b_rows, (1, 1))


def kernel(q_emb, d_emb, offset):
    b, k = q_emb.shape
    n = d_emb.shape[0]
    bn = 1024
    n_col_blocks = n // bn

    offset = jnp.asarray(offset, dtype=jnp.int32)
    q_emb = q_emb.astype(jnp.bfloat16)
    d_emb = d_emb.astype(jnp.bfloat16)
    d_pos = jax.lax.dynamic_slice(d_emb, (offset, 0), (b, k))

    body = functools.partial(_body, n_col_blocks=n_col_blocks, b_rows=b)
    out = pl.pallas_call(
        body,
        grid=(n_col_blocks,),
        in_specs=[
            pl.BlockSpec((b, k), lambda c: (0, 0)),
            pl.BlockSpec((bn, k), lambda c: (c, 0)),
            pl.BlockSpec((b, k), lambda c: (0, 0)),
        ],
        out_specs=pl.BlockSpec((1, 1), lambda c: (0, 0)),
        out_shape=jax.ShapeDtypeStruct((1, 1), jnp.float32),
        scratch_shapes=[
            pltpu.VMEM((b, 1), jnp.float32),
            pltpu.VMEM((b, 1), jnp.float32),
            pltpu.VMEM((b, 1), jnp.float32),
        ],
    )(q_emb, d_emb, d_pos)
    return out[0, 0]


# speedup vs baseline: 1.3858x; 1.3858x over previous
import functools
import math

import jax
import jax.numpy as jnp
from jax.experimental import pallas as pl
from jax.experimental.pallas import tpu as pltpu

TEMPERATURE = 0.02
FILTER_THRESHOLD = 0.95
FILTER_FACTOR = 0.5
SCALE = 1.0 / TEMPERATURE
LOG2E = math.log2(math.e)


def _body(q_ref, d_ref, dpos_ref, out_ref,
          qs_ref, pos_ref, m_ref, l_ref, *, n_col_blocks, b_rows):
    c = pl.program_id(0)

    @pl.when(c == 0)
    def _init():
        q = q_ref[...]
        pos_ref[...] = (
            jnp.sum(q * dpos_ref[...], axis=1, keepdims=True)
            * (SCALE * LOG2E)
        )
        qs_ref[...] = (q * (SCALE * LOG2E)).astype(jnp.bfloat16)
        m_ref[...] = jnp.full((b_rows, 1), -jnp.inf, dtype=jnp.float32)
        l_ref[...] = jnp.zeros((b_rows, 1), dtype=jnp.float32)

    s = jax.lax.dot_general(
        qs_ref[...], d_ref[...].astype(jnp.bfloat16),
        dimension_numbers=(((1,), (1,)), ((), ())),
        preferred_element_type=jnp.float32,
    )

    thresh = FILTER_THRESHOLD * pos_ref[...]
    s = jnp.where(s > thresh, s * FILTER_FACTOR, s)

    m_prev = m_ref[...]
    m_cur = jnp.maximum(m_prev, jnp.max(s, axis=1, keepdims=True))
    l_ref[...] = (
        l_ref[...] * jnp.exp2(m_prev - m_cur)
        + jnp.sum(jnp.exp2(s - m_cur), axis=1, keepdims=True)
    )
    m_ref[...] = m_cur

    @pl.when(c == n_col_blocks - 1)
    def _final():
        pos = pos_ref[...]
        m_run = m_ref[...]
        l_run = l_ref[...]
        m_true = jnp.maximum(m_run, pos)
        corr = jnp.where(
            pos > 0.0,
            jnp.exp2(pos - m_true) - jnp.exp2(FILTER_FACTOR * pos - m_true),
            0.0,
        )
        l_true = l_run * jnp.exp2(m_run - m_true) + corr
        lse = m_true + jnp.log2(l_true)
        out_ref[...] = jnp.reshape(
            -jnp.sum(pos - lse) / (LOG2E * b_rows), (1, 1)
        )


def kernel(q_emb, d_emb, offset):
    b, k = q_emb.shape
    n = d_emb.shape[0]
    bn = 1024
    n_col_blocks = n // bn

    offset = jnp.asarray(offset, dtype=jnp.int32)
    d_pos = jax.lax.dynamic_slice(d_emb, (offset, 0), (b, k))

    body = functools.partial(_body, n_col_blocks=n_col_blocks, b_rows=b)
    out = pl.pallas_call(
        body,
        grid=(n_col_blocks,),
        in_specs=[
            pl.BlockSpec((b, k), lambda c: (0, 0)),
            pl.BlockSpec((bn, k), lambda c: (c, 0)),
            pl.BlockSpec((b, k), lambda c: (0, 0)),
        ],
        out_specs=pl.BlockSpec((1, 1), lambda c: (0, 0)),
        out_shape=jax.ShapeDtypeStruct((1, 1), jnp.float32),
        scratch_shapes=[
            pltpu.VMEM((b, k), jnp.bfloat16),
            pltpu.VMEM((b, 1), jnp.float32),
            pltpu.VMEM((b, 1), jnp.float32),
            pltpu.VMEM((b, 1), jnp.float32),
        ],
    )(q_emb, d_emb, d_pos)
    return out[0, 0]
